# Initial kernel scaffold; baseline (speedup 1.0000x reference)
import jax
import jax.numpy as jnp
from jax.experimental import pallas as pl


def kernel(x, edge_index, edge_weight, W1, b1, W2, b2):
    raise NotImplementedError("write your pallas kernel here")


# speedup vs baseline: 6.5114x; 6.5114x over previous
import functools

import jax
import jax.numpy as jnp
from jax import lax
from jax.experimental import pallas as pl
from jax.experimental.pallas import tpu as pltpu
from jax.experimental.pallas import tpu_sc as plsc

N = 10000
NPAD = 10240
F_IN = 256
HID = 16
C = 64

NW = 32
NCHUNK = 42
K = 128
EPAD = NW * NCHUNK * K
RPT = NPAD // 16
RCHUNK = RPT // K

_MESH = plsc.VectorSubcoreMesh(core_axis_name="c", subcore_axis_name="s")


def _zero_rows(buf):
    zero = jnp.zeros((16,), jnp.float32)

    def zbody(i, carry):
        for d in range(8):
            buf[i, pl.ds(d * 16, 16)] = zero
        return carry

    lax.fori_loop(0, K, zbody, 0)


def _acc_init(buf, acc, s):
    def ibody(t, carry):
        pltpu.sync_copy(buf, acc.at[pl.ds(s * RPT + t * K, K)])
        return carry

    lax.fori_loop(0, RCHUNK, ibody, 0)


def _sc_deg_body(col_hbm, ew16_hbm, out_hbm, cidx, ewk, msg, acc, sem):
    c = lax.axis_index("c")
    s = lax.axis_index("s")
    wid = s * 2 + c
    pltpu.sync_copy(col_hbm.at[wid], cidx)
    _zero_rows(msg)
    _acc_init(msg, acc, s)
    plsc.subcore_barrier()

    for ci in range(NCHUNK):
        pltpu.async_copy(ew16_hbm.at[wid, ci], ewk, sem).wait()

        def ebody(j, carry):
            msg[j, pl.ds(0, 16)] = ewk[j]
            return carry

        lax.fori_loop(0, K, ebody, 0)
        pltpu.sync_copy(msg, acc.at[cidx.at[ci]], add=True)

    plsc.subcore_barrier()

    def rbody(t, carry):
        pltpu.sync_copy(acc.at[pl.ds(s * RPT + t * K, K)], msg)
        pltpu.sync_copy(msg, out_hbm.at[c, pl.ds(s * RPT + t * K, K)])
        return carry

    lax.fori_loop(0, RCHUNK, rbody, 0)


def _sc_layer_body(fd, tab_hbm, row_hbm, col_hbm, ew16_hbm, out_hbm,
                   ridx, cidx, ewk, rows, acc, sem, sem2):
    c = lax.axis_index("c")
    s = lax.axis_index("s")
    wid = s * 2 + c
    pltpu.sync_copy(row_hbm.at[wid], ridx)
    pltpu.sync_copy(col_hbm.at[wid], cidx)
    _zero_rows(rows)
    _acc_init(rows, acc, s)
    plsc.subcore_barrier()

    for ci in range(NCHUNK):
        ca = pltpu.async_copy(tab_hbm.at[ridx.at[ci]], rows, sem)
        cb = pltpu.async_copy(ew16_hbm.at[wid, ci], ewk, sem2)
        ca.wait()
        cb.wait()

        def ebody(j, carry):
            w = ewk[j]
            for d in range(fd):
                rows[j, pl.ds(d * 16, 16)] = rows[j, pl.ds(d * 16, 16)] * w
            return carry

        lax.fori_loop(0, K, ebody, 0)
        pltpu.sync_copy(rows, acc.at[cidx.at[ci]], add=True)

    plsc.subcore_barrier()

    def rbody(t, carry):
        pltpu.sync_copy(acc.at[pl.ds(s * RPT + t * K, K)], rows)
        pltpu.sync_copy(rows, out_hbm.at[c, pl.ds(s * RPT + t * K, K)])
        return carry

    lax.fori_loop(0, RCHUNK, rbody, 0)


def _make_deg_call():
    return pl.kernel(
        _sc_deg_body,
        out_type=jax.ShapeDtypeStruct((2, NPAD, 128), jnp.float32),
        mesh=_MESH,
        scratch_types=[
            pltpu.VMEM((NCHUNK, K), jnp.int32),
            pltpu.VMEM((K, 16), jnp.float32),
            pltpu.VMEM((K, 128), jnp.float32),
            pltpu.VMEM_SHARED((NPAD, 128), jnp.float32),
            pltpu.SemaphoreType.DMA,
        ],
    )


def _make_layer_call(fd):
    return pl.kernel(
        functools.partial(_sc_layer_body, fd),
        out_type=jax.ShapeDtypeStruct((2, NPAD, 128), jnp.float32),
        mesh=_MESH,
        scratch_types=[
            pltpu.VMEM((NCHUNK, K), jnp.int32),
            pltpu.VMEM((NCHUNK, K), jnp.int32),
            pltpu.VMEM((K, 16), jnp.float32),
            pltpu.VMEM((K, 128), jnp.float32),
            pltpu.VMEM_SHARED((NPAD, 128), jnp.float32),
            pltpu.SemaphoreType.DMA,
            pltpu.SemaphoreType.DMA,
        ],
    )


BLK = 1280
GRID = NPAD // BLK


def _tc1_body(x_ref, w1_ref, degp_ref, h1p_ref, dis_ref):
    deg = degp_ref[0, :, 0:HID] + degp_ref[1, :, 0:HID]
    dis = jnp.where(deg > 0.0, lax.rsqrt(deg), 0.0)
    h = jnp.dot(x_ref[...], w1_ref[...], preferred_element_type=jnp.float32)
    h1p_ref[:, 0:HID] = h * dis
    h1p_ref[:, HID:] = jnp.zeros((BLK, 128 - HID), jnp.float32)
    dis_ref[...] = dis


def _tc1(xp, W1, degp):
    return pl.pallas_call(
        _tc1_body,
        grid=(GRID,),
        in_specs=[
            pl.BlockSpec((BLK, F_IN), lambda i: (i, 0)),
            pl.BlockSpec((F_IN, HID), lambda i: (0, 0)),
            pl.BlockSpec((2, BLK, 128), lambda i: (0, i, 0)),
        ],
        out_specs=[
            pl.BlockSpec((BLK, 128), lambda i: (i, 0)),
            pl.BlockSpec((BLK, HID), lambda i: (i, 0)),
        ],
        out_shape=[
            jax.ShapeDtypeStruct((NPAD, 128), jnp.float32),
            jax.ShapeDtypeStruct((NPAD, HID), jnp.float32),
        ],
    )(xp, W1, degp)


def _tc2_body(s1_ref, dis_ref, b1_ref, w2_ref, h2p_ref):
    dis = dis_ref[...]
    agg = dis * (s1_ref[0, :, 0:HID] + s1_ref[1, :, 0:HID]) + b1_ref[...]
    h = jnp.maximum(agg, 0.0)
    h2 = jnp.dot(h, w2_ref[...], preferred_element_type=jnp.float32)
    disb = jnp.broadcast_to(dis[:, 0:1], (BLK, C))
    h2p_ref[:, 0:C] = h2 * disb
    h2p_ref[:, C:] = jnp.zeros((BLK, 128 - C), jnp.float32)


def _tc2(s1, dis16, b1, W2):
    return pl.pallas_call(
        _tc2_body,
        grid=(GRID,),
        in_specs=[
            pl.BlockSpec((2, BLK, 128), lambda i: (0, i, 0)),
            pl.BlockSpec((BLK, HID), lambda i: (i, 0)),
            pl.BlockSpec((1, HID), lambda i: (0, 0)),
            pl.BlockSpec((HID, C), lambda i: (0, 0)),
        ],
        out_specs=pl.BlockSpec((BLK, 128), lambda i: (i, 0)),
        out_shape=jax.ShapeDtypeStruct((NPAD, 128), jnp.float32),
    )(s1, dis16, b1, W2)


def _tc3_body(s2_ref, dis_ref, b2_ref, out_ref):
    disb = jnp.broadcast_to(dis_ref[:, 0:1], (BLK, C))
    z = disb * (s2_ref[0, :, 0:C] + s2_ref[1, :, 0:C]) + b2_ref[...]
    m = jnp.max(z, axis=1, keepdims=True)
    zm = z - m
    lse = jnp.log(jnp.sum(jnp.exp(zm), axis=1, keepdims=True))
    out_ref[...] = zm - lse


def _tc3(s2, dis16, b2):
    return pl.pallas_call(
        _tc3_body,
        grid=(GRID,),
        in_specs=[
            pl.BlockSpec((2, BLK, 128), lambda i: (0, i, 0)),
            pl.BlockSpec((BLK, HID), lambda i: (i, 0)),
            pl.BlockSpec((1, C), lambda i: (0, 0)),
        ],
        out_specs=pl.BlockSpec((BLK, C), lambda i: (i, 0)),
        out_shape=jax.ShapeDtypeStruct((NPAD, C), jnp.float32),
    )(s2, dis16, b2)


def kernel(x, edge_index, edge_weight, W1, b1, W2, b2):
    loop = jnp.arange(N, dtype=jnp.int32)
    row = jnp.concatenate([edge_index[0].astype(jnp.int32), loop])
    col = jnp.concatenate([edge_index[1].astype(jnp.int32), loop])
    ew = jnp.concatenate([edge_weight.astype(jnp.float32),
                          jnp.ones((N,), jnp.float32)])
    pad = EPAD - row.shape[0]
    row = jnp.pad(row, (0, pad)).reshape(NW, NCHUNK, K)
    col = jnp.pad(col, (0, pad)).reshape(NW, NCHUNK, K)
    ew16 = jnp.broadcast_to(
        jnp.pad(ew, (0, pad)).reshape(NW, NCHUNK, K, 1), (NW, NCHUNK, K, 16)
    )
    xp = jnp.pad(x, ((0, NPAD - N), (0, 0)))

    degp = _make_deg_call()(col, ew16)
    h1p, dis16 = _tc1(xp, W1, degp)
    s1 = _make_layer_call(1)(h1p, row, col, ew16)
    h2p = _tc2(s1, dis16, b1.reshape(1, HID), W2)
    s2 = _make_layer_call(4)(h2p, row, col, ew16)
    out = _tc3(s2, dis16, b2.reshape(1, C))
    return out[:N]


# speedup vs baseline: 9.9476x; 1.5277x over previous
import functools

import jax
import jax.numpy as jnp
from jax import lax
from jax.experimental import pallas as pl
from jax.experimental.pallas import tpu as pltpu
from jax.experimental.pallas import tpu_sc as plsc

N = 10000
NPAD = 10240
F_IN = 256
HID = 16
C = 64

NW = 32
NCHUNK = 42
K = 128
EPAD = NW * NCHUNK * K
RPT = NPAD // 16
RCHUNK = RPT // K

_MESH = plsc.VectorSubcoreMesh(core_axis_name="c", subcore_axis_name="s")


def _zero_rows(buf):
    zero = jnp.zeros((16,), jnp.float32)

    def zbody(i, carry):
        for d in range(8):
            buf[i, pl.ds(d * 16, 16)] = zero
        return carry

    lax.fori_loop(0, K, zbody, 0)


def _acc_init(buf, acc, s):
    def ibody(t, carry):
        pltpu.sync_copy(buf, acc.at[pl.ds(s * RPT + t * K, K)])
        return carry

    lax.fori_loop(0, RCHUNK, ibody, 0)


def _sc_deg_body(col_hbm, ew16_hbm, out_hbm, cidx, ewk0, ewk1, msg0, msg1,
                 acc, es0, es1, ss0, ss1):
    c = lax.axis_index("c")
    s = lax.axis_index("s")
    wid = s * 2 + c
    pltpu.sync_copy(col_hbm.at[wid], cidx)
    _zero_rows(msg0)
    _zero_rows(msg1)
    _acc_init(msg0, acc, s)
    plsc.subcore_barrier()

    ewk = [ewk0, ewk1]
    msg = [msg0, msg1]
    esem = [es0, es1]
    ssem = [ss0, ss1]

    def splat(b):
        @plsc.parallel_loop(0, 16, unroll=2)
        def ebody(jo):
            for a in range(8):
                msg[b][jo * 8 + a, pl.ds(0, 16)] = ewk[b][jo, pl.ds(a * 16, 16)]

    ed = [pltpu.async_copy(ew16_hbm.at[wid, b], ewk[b], esem[b])
          for b in (0, 1)]

    def lbody(i, carry):
        g = 2 * i
        sd = []
        for b in (0, 1):
            ed[b].wait()
            splat(b)
            sd.append(pltpu.async_copy(msg[b], acc.at[cidx.at[g + b]],
                                       ssem[b], add=True))
        for b in (0, 1):
            sd[b].wait()
            pltpu.async_copy(ew16_hbm.at[wid, g + 2 + b], ewk[b], esem[b])
        return carry

    lax.fori_loop(0, (NCHUNK - 2) // 2, lbody, 0)
    for b in (0, 1):
        ed[b].wait()
        splat(b)
        pltpu.async_copy(msg[b], acc.at[cidx.at[NCHUNK - 2 + b]],
                         ssem[b], add=True).wait()
    plsc.subcore_barrier()

    def rbody(t, carry):
        pltpu.sync_copy(acc.at[pl.ds(s * RPT + t * K, K)], msg0)
        pltpu.sync_copy(msg0, out_hbm.at[c, pl.ds(s * RPT + t * K, K)])
        return carry

    lax.fori_loop(0, RCHUNK, rbody, 0)


def _sc_layer_body(fd, tab_hbm, row_hbm, col_hbm, ew16_hbm, out_hbm,
                   ridx, cidx, ewk0, ewk1, rows0, rows1, acc,
                   gs0, gs1, es0, es1, ss0, ss1):
    c = lax.axis_index("c")
    s = lax.axis_index("s")
    wid = s * 2 + c
    pltpu.sync_copy(row_hbm.at[wid], ridx)
    pltpu.sync_copy(col_hbm.at[wid], cidx)
    _zero_rows(rows0)
    _acc_init(rows0, acc, s)
    plsc.subcore_barrier()

    ewk = [ewk0, ewk1]
    rows = [rows0, rows1]
    gsem = [gs0, gs1]
    esem = [es0, es1]
    ssem = [ss0, ss1]

    def scale(b):
        @plsc.parallel_loop(0, 16, unroll=2)
        def ebody(jo):
            for a in range(8):
                w = ewk[b][jo, pl.ds(a * 16, 16)]
                for d in range(fd):
                    rows[b][jo * 8 + a, pl.ds(d * 16, 16)] = (
                        rows[b][jo * 8 + a, pl.ds(d * 16, 16)] * w)

    gd = [pltpu.async_copy(tab_hbm.at[ridx.at[b]], rows[b], gsem[b])
          for b in (0, 1)]
    ed = [pltpu.async_copy(ew16_hbm.at[wid, b], ewk[b], esem[b])
          for b in (0, 1)]

    def lbody(i, carry):
        g = 2 * i
        sd = []
        for b in (0, 1):
            gd[b].wait()
            ed[b].wait()
            scale(b)
            sd.append(pltpu.async_copy(rows[b], acc.at[cidx.at[g + b]],
                                       ssem[b], add=True))
        for b in (0, 1):
            sd[b].wait()
            pltpu.async_copy(tab_hbm.at[ridx.at[g + 2 + b]], rows[b], gsem[b])
            pltpu.async_copy(ew16_hbm.at[wid, g + 2 + b], ewk[b], esem[b])
        return carry

    lax.fori_loop(0, (NCHUNK - 2) // 2, lbody, 0)
    for b in (0, 1):
        gd[b].wait()
        ed[b].wait()
        scale(b)
        pltpu.async_copy(rows[b], acc.at[cidx.at[NCHUNK - 2 + b]],
                         ssem[b], add=True).wait()
    plsc.subcore_barrier()

    def rbody(t, carry):
        pltpu.sync_copy(acc.at[pl.ds(s * RPT + t * K, K)], rows0)
        pltpu.sync_copy(rows0, out_hbm.at[c, pl.ds(s * RPT + t * K, K)])
        return carry

    lax.fori_loop(0, RCHUNK, rbody, 0)


def _make_deg_call():
    return pl.kernel(
        _sc_deg_body,
        out_type=jax.ShapeDtypeStruct((2, NPAD, 128), jnp.float32),
        mesh=_MESH,
        scratch_types=[
            pltpu.VMEM((NCHUNK, K), jnp.int32),
            pltpu.VMEM((16, 128), jnp.float32),
            pltpu.VMEM((16, 128), jnp.float32),
            pltpu.VMEM((K, 128), jnp.float32),
            pltpu.VMEM((K, 128), jnp.float32),
            pltpu.VMEM_SHARED((NPAD, 128), jnp.float32),
            pltpu.SemaphoreType.DMA,
            pltpu.SemaphoreType.DMA,
            pltpu.SemaphoreType.DMA,
            pltpu.SemaphoreType.DMA,
        ],
    )


def _make_layer_call(fd):
    return pl.kernel(
        functools.partial(_sc_layer_body, fd),
        out_type=jax.ShapeDtypeStruct((2, NPAD, 128), jnp.float32),
        mesh=_MESH,
        scratch_types=[
            pltpu.VMEM((NCHUNK, K), jnp.int32),
            pltpu.VMEM((NCHUNK, K), jnp.int32),
            pltpu.VMEM((16, 128), jnp.float32),
            pltpu.VMEM((16, 128), jnp.float32),
            pltpu.VMEM((K, 128), jnp.float32),
            pltpu.VMEM((K, 128), jnp.float32),
            pltpu.VMEM_SHARED((NPAD, 128), jnp.float32),
            pltpu.SemaphoreType.DMA,
            pltpu.SemaphoreType.DMA,
            pltpu.SemaphoreType.DMA,
            pltpu.SemaphoreType.DMA,
            pltpu.SemaphoreType.DMA,
            pltpu.SemaphoreType.DMA,
        ],
    )


BLK = 1280
GRID = NPAD // BLK


def _tc1_body(x_ref, w1_ref, degp_ref, h1p_ref, dis_ref):
    deg = degp_ref[0, :, 0:HID] + degp_ref[1, :, 0:HID]
    dis = jnp.where(deg > 0.0, lax.rsqrt(deg), 0.0)
    h = jnp.dot(x_ref[...], w1_ref[...], preferred_element_type=jnp.float32)
    h1p_ref[:, 0:HID] = h * dis
    h1p_ref[:, HID:] = jnp.zeros((BLK, 128 - HID), jnp.float32)
    dis_ref[...] = dis


def _tc1(xp, W1, degp):
    return pl.pallas_call(
        _tc1_body,
        grid=(GRID,),
        in_specs=[
            pl.BlockSpec((BLK, F_IN), lambda i: (i, 0)),
            pl.BlockSpec((F_IN, HID), lambda i: (0, 0)),
            pl.BlockSpec((2, BLK, 128), lambda i: (0, i, 0)),
        ],
        out_specs=[
            pl.BlockSpec((BLK, 128), lambda i: (i, 0)),
            pl.BlockSpec((BLK, HID), lambda i: (i, 0)),
        ],
        out_shape=[
            jax.ShapeDtypeStruct((NPAD, 128), jnp.float32),
            jax.ShapeDtypeStruct((NPAD, HID), jnp.float32),
        ],
    )(xp, W1, degp)


def _tc2_body(s1_ref, dis_ref, b1_ref, w2_ref, h2p_ref):
    dis = dis_ref[...]
    agg = dis * (s1_ref[0, :, 0:HID] + s1_ref[1, :, 0:HID]) + b1_ref[...]
    h = jnp.maximum(agg, 0.0)
    h2 = jnp.dot(h, w2_ref[...], preferred_element_type=jnp.float32)
    disb = jnp.broadcast_to(dis[:, 0:1], (BLK, C))
    h2p_ref[:, 0:C] = h2 * disb
    h2p_ref[:, C:] = jnp.zeros((BLK, 128 - C), jnp.float32)


def _tc2(s1, dis16, b1, W2):
    return pl.pallas_call(
        _tc2_body,
        grid=(GRID,),
        in_specs=[
            pl.BlockSpec((2, BLK, 128), lambda i: (0, i, 0)),
            pl.BlockSpec((BLK, HID), lambda i: (i, 0)),
            pl.BlockSpec((1, HID), lambda i: (0, 0)),
            pl.BlockSpec((HID, C), lambda i: (0, 0)),
        ],
        out_specs=pl.BlockSpec((BLK, 128), lambda i: (i, 0)),
        out_shape=jax.ShapeDtypeStruct((NPAD, 128), jnp.float32),
    )(s1, dis16, b1, W2)


def _tc3_body(s2_ref, dis_ref, b2_ref, out_ref):
    disb = jnp.broadcast_to(dis_ref[:, 0:1], (BLK, C))
    z = disb * (s2_ref[0, :, 0:C] + s2_ref[1, :, 0:C]) + b2_ref[...]
    m = jnp.max(z, axis=1, keepdims=True)
    zm = z - m
    lse = jnp.log(jnp.sum(jnp.exp(zm), axis=1, keepdims=True))
    out_ref[...] = zm - lse


def _tc3(s2, dis16, b2):
    return pl.pallas_call(
        _tc3_body,
        grid=(GRID,),
        in_specs=[
            pl.BlockSpec((2, BLK, 128), lambda i: (0, i, 0)),
            pl.BlockSpec((BLK, HID), lambda i: (i, 0)),
            pl.BlockSpec((1, C), lambda i: (0, 0)),
        ],
        out_specs=pl.BlockSpec((BLK, C), lambda i: (i, 0)),
        out_shape=jax.ShapeDtypeStruct((NPAD, C), jnp.float32),
    )(s2, dis16, b2)


def kernel(x, edge_index, edge_weight, W1, b1, W2, b2):
    loop = jnp.arange(N, dtype=jnp.int32)
    row = jnp.concatenate([edge_index[0].astype(jnp.int32), loop])
    col = jnp.concatenate([edge_index[1].astype(jnp.int32), loop])
    ew = jnp.concatenate([edge_weight.astype(jnp.float32),
                          jnp.ones((N,), jnp.float32)])
    pad = EPAD - row.shape[0]
    row = jnp.pad(row, (0, pad)).reshape(NW, NCHUNK, K)
    col = jnp.pad(col, (0, pad)).reshape(NW, NCHUNK, K)
    ew16 = jnp.broadcast_to(
        jnp.pad(ew, (0, pad)).reshape(NW, NCHUNK, K, 1), (NW, NCHUNK, K, 16)
    ).reshape(NW, NCHUNK, 16, 128)
    xp = jnp.pad(x, ((0, NPAD - N), (0, 0)))

    degp = _make_deg_call()(col, ew16)
    h1p, dis16 = _tc1(xp, W1, degp)
    s1 = _make_layer_call(1)(h1p, row, col, ew16)
    h2p = _tc2(s1, dis16, b1.reshape(1, HID), W2)
    s2 = _make_layer_call(4)(h2p, row, col, ew16)
    out = _tc3(s2, dis16, b2.reshape(1, C))
    return out[:N]
